# baseline (speedup 1.0000x reference)
import functools

import jax
import jax.numpy as jnp
from jax.experimental import pallas as pl
from jax.experimental.pallas import tpu as pltpu

_BN_EPS = 1e-5
_VMEM_LIMIT = 100 * 1024 * 1024


def _pad_hw(t):
    H, W, C = t.shape
    zr = jnp.zeros((1, W, C), t.dtype)
    t = jnp.concatenate([zr, t, zr], axis=0)
    zc = jnp.zeros((H + 2, 1, C), t.dtype)
    return jnp.concatenate([zc, t, zc], axis=1)


def _patches(tp, H, W):
    taps = [tp[i:i + H, j:j + W, :] for i in range(3) for j in range(3)]
    return jnp.concatenate(taps, axis=2).reshape(H * W, 9 * tp.shape[-1])


def _stats_rows(y, st_shape):
    s = jnp.sum(y, axis=0, keepdims=True)
    q = jnp.sum(y * y, axis=0, keepdims=True)
    row = jax.lax.broadcasted_iota(jnp.int32, st_shape, 0)
    return jnp.where(row == 0, s, jnp.where(row == 1, q, 0.0))


def _scale_shift(P, st, g, b, inv_m):
    tot = jnp.sum(st[:, 0, :], axis=0, keepdims=True)
    totsq = jnp.sum(st[:, 1, :], axis=0, keepdims=True)
    C = tot.shape[1] // P
    tot = sum(tot[:, k * C:(k + 1) * C] for k in range(P))
    totsq = sum(totsq[:, k * C:(k + 1) * C] for k in range(P))
    mean = tot * inv_m
    var = jnp.maximum(totsq * inv_m - mean * mean, 0.0)
    inv = jax.lax.rsqrt(var + _BN_EPS)
    scale = g * inv
    shift = b - mean * scale
    if P > 1:
        scale = jnp.concatenate([scale] * P, axis=1)
        shift = jnp.concatenate([shift] * P, axis=1)
    return scale, shift


def _conv1_kernel(H, W, x_ref, w_ref, y_ref, st_ref):
    P, C = x_ref.shape[0], x_ref.shape[1]
    x2 = x_ref[...].reshape(P * C, H * W)
    t = jnp.transpose(x2)
    x3 = t.reshape(H, W, P * C)
    p = _patches(_pad_hw(x3), H, W)
    y = jnp.dot(p, w_ref[...], preferred_element_type=jnp.float32)
    y_ref[0] = y.astype(y_ref.dtype)
    st_ref[0] = _stats_rows(y, st_ref.shape[1:])


def _affine_conv2_kernel(P, H, W, inv_m, y_ref, st1_ref, g_ref, b_ref, w_ref,
                         y2_ref, st_ref):
    scale, shift = _scale_shift(P, st1_ref[...], g_ref[...], b_ref[...], inv_m)
    t = y_ref[0] * scale + shift
    t3 = t.reshape(H, W, y_ref.shape[2]).astype(jnp.bfloat16)
    p = _patches(_pad_hw(t3), H, W)
    y = jnp.dot(p, w_ref[...], preferred_element_type=jnp.float32)
    y2_ref[0] = y.astype(y2_ref.dtype)
    st_ref[0] = _stats_rows(y, st_ref.shape[1:])


def _affine_out_kernel(P, H, W, inv_m, y_ref, st2_ref, g_ref, b_ref, o_ref):
    scale, shift = _scale_shift(P, st2_ref[...], g_ref[...], b_ref[...], inv_m)
    t = y_ref[0] * scale + shift
    tT = jnp.transpose(t)
    o_ref[...] = tT.reshape(P, tT.shape[0] // P, H * W)


def _w_block(w, P):
    Co, Ci, kh, kw = w.shape
    wm = jnp.transpose(w, (2, 3, 1, 0)).reshape(kh * kw, Ci, Co)
    z = jnp.zeros_like(wm)
    rows = []
    for a in range(P):
        rows.append(jnp.concatenate([wm if a == b else z for b in range(P)],
                                    axis=2))
    wb = jnp.concatenate(rows, axis=1)
    return wb.reshape(kh * kw * P * Ci, P * Co).astype(jnp.bfloat16)


@jax.jit
def _forward(x, w1, g1, b1, w2, g2, b2):
    N, C, H, W = x.shape
    HW = H * W
    Co1, Co2 = w1.shape[0], w2.shape[0]
    P = 2 if N % 2 == 0 else 1
    NP = N // P
    grid = (2, NP // 2) if NP % 2 == 0 else (1, NP)
    npc = grid[1]
    params = pltpu.CompilerParams(
        dimension_semantics=("parallel", "arbitrary"),
        vmem_limit_bytes=_VMEM_LIMIT)

    def img(c, i):
        return c * npc + i

    xd = x.reshape(N, C, HW).astype(jnp.bfloat16)
    inv_m = 1.0 / (N * HW)
    g1r = g1.astype(jnp.float32).reshape(1, Co1)
    b1r = b1.astype(jnp.float32).reshape(1, Co1)
    g2r = g2.astype(jnp.float32).reshape(1, Co2)
    b2r = b2.astype(jnp.float32).reshape(1, Co2)

    y1, st1 = pl.pallas_call(
        functools.partial(_conv1_kernel, H, W),
        out_shape=(jax.ShapeDtypeStruct((NP, HW, P * Co1), jnp.float32),
                   jax.ShapeDtypeStruct((NP, 8, P * Co1), jnp.float32)),
        grid=grid,
        in_specs=[pl.BlockSpec((P, C, HW), lambda c, i: (img(c, i), 0, 0)),
                  pl.BlockSpec((9 * P * C, P * Co1), lambda c, i: (0, 0))],
        out_specs=(pl.BlockSpec((1, HW, P * Co1),
                                lambda c, i: (img(c, i), 0, 0)),
                   pl.BlockSpec((1, 8, P * Co1),
                                lambda c, i: (img(c, i), 0, 0))),
        compiler_params=params,
    )(xd, _w_block(w1, P))

    y2, st2 = pl.pallas_call(
        functools.partial(_affine_conv2_kernel, P, H, W, inv_m),
        out_shape=(jax.ShapeDtypeStruct((NP, HW, P * Co2), jnp.float32),
                   jax.ShapeDtypeStruct((NP, 8, P * Co2), jnp.float32)),
        grid=grid,
        in_specs=[pl.BlockSpec((1, HW, P * Co1), lambda c, i: (img(c, i), 0, 0)),
                  pl.BlockSpec((NP, 8, P * Co1), lambda c, i: (0, 0, 0)),
                  pl.BlockSpec((1, Co1), lambda c, i: (0, 0)),
                  pl.BlockSpec((1, Co1), lambda c, i: (0, 0)),
                  pl.BlockSpec((9 * P * Co1, P * Co2), lambda c, i: (0, 0))],
        out_specs=(pl.BlockSpec((1, HW, P * Co2),
                                lambda c, i: (img(c, i), 0, 0)),
                   pl.BlockSpec((1, 8, P * Co2),
                                lambda c, i: (img(c, i), 0, 0))),
        compiler_params=params,
    )(y1, st1, g1r, b1r, _w_block(w2, P))

    out = pl.pallas_call(
        functools.partial(_affine_out_kernel, P, H, W, inv_m),
        out_shape=jax.ShapeDtypeStruct((N, Co2, HW), jnp.float32),
        grid=grid,
        in_specs=[pl.BlockSpec((1, HW, P * Co2), lambda c, i: (img(c, i), 0, 0)),
                  pl.BlockSpec((NP, 8, P * Co2), lambda c, i: (0, 0, 0)),
                  pl.BlockSpec((1, Co2), lambda c, i: (0, 0)),
                  pl.BlockSpec((1, Co2), lambda c, i: (0, 0))],
        out_specs=pl.BlockSpec((P, Co2, HW), lambda c, i: (img(c, i), 0, 0)),
        compiler_params=params,
    )(y2, st2, g2r, b2r)

    return out.reshape(N, Co2, H, W)


def kernel(x, w1, g1, b1, w2, g2, b2):
    return _forward(x, w1, g1, b1, w2, g2, b2)


# speedup vs baseline: 1.0894x; 1.0894x over previous
import functools

import jax
import jax.numpy as jnp
from jax.experimental import pallas as pl
from jax.experimental.pallas import tpu as pltpu

_BN_EPS = 1e-5
_VMEM_LIMIT = 100 * 1024 * 1024


def _pad_hw(t):
    H, W, C = t.shape
    zr = jnp.zeros((1, W, C), t.dtype)
    t = jnp.concatenate([zr, t, zr], axis=0)
    zc = jnp.zeros((H + 2, 1, C), t.dtype)
    return jnp.concatenate([zc, t, zc], axis=1)


def _patches(tp, H, W):
    taps = [tp[i:i + H, j:j + W, :] for i in range(3) for j in range(3)]
    return jnp.concatenate(taps, axis=2).reshape(H * W, 9 * tp.shape[-1])


def _stats_rows(y, st_shape):
    s = jnp.sum(y, axis=0, keepdims=True)
    q = jnp.sum(y * y, axis=0, keepdims=True)
    row = jax.lax.broadcasted_iota(jnp.int32, st_shape, 0)
    return jnp.where(row == 0, s, jnp.where(row == 1, q, 0.0))


def _scale_shift(P, st, g, b, inv_m):
    tot = jnp.sum(st[:, 0, :], axis=0, keepdims=True)
    totsq = jnp.sum(st[:, 1, :], axis=0, keepdims=True)
    C = tot.shape[1] // P
    tot = sum(tot[:, k * C:(k + 1) * C] for k in range(P))
    totsq = sum(totsq[:, k * C:(k + 1) * C] for k in range(P))
    mean = tot * inv_m
    var = jnp.maximum(totsq * inv_m - mean * mean, 0.0)
    inv = jax.lax.rsqrt(var + _BN_EPS)
    scale = g * inv
    shift = b - mean * scale
    if P > 1:
        scale = jnp.concatenate([scale] * P, axis=1)
        shift = jnp.concatenate([shift] * P, axis=1)
    return scale, shift


def _conv1_kernel(H, W, x_ref, w_ref, y_ref, st_ref):
    P, C = x_ref.shape[0], x_ref.shape[1]
    x2 = x_ref[...].reshape(P * C, H * W)
    t = jnp.transpose(x2)
    x3 = t.reshape(H, W, P * C).astype(jnp.bfloat16)
    p = _patches(_pad_hw(x3), H, W)
    y = jnp.dot(p, w_ref[...], preferred_element_type=jnp.float32)
    y_ref[0] = y.astype(y_ref.dtype)
    st_ref[0] = _stats_rows(y, st_ref.shape[1:])


def _affine_conv2_kernel(P, H, W, inv_m, y_ref, st1_ref, g_ref, b_ref, w_ref,
                         y2_ref, st_ref):
    scale, shift = _scale_shift(P, st1_ref[...], g_ref[...], b_ref[...], inv_m)
    t = y_ref[0] * scale + shift
    t3 = t.reshape(H, W, y_ref.shape[2]).astype(jnp.bfloat16)
    p = _patches(_pad_hw(t3), H, W)
    y = jnp.dot(p, w_ref[...], preferred_element_type=jnp.float32)
    y2_ref[0] = y.astype(y2_ref.dtype)
    st_ref[0] = _stats_rows(y, st_ref.shape[1:])


def _affine_out_kernel(P, H, W, inv_m, y_ref, st2_ref, g_ref, b_ref, o_ref):
    scale, shift = _scale_shift(P, st2_ref[...], g_ref[...], b_ref[...], inv_m)
    t = y_ref[0] * scale + shift
    tT = jnp.transpose(t)
    o_ref[...] = tT.reshape(P, tT.shape[0] // P, H * W)


def _w_block(w, P):
    Co, Ci, kh, kw = w.shape
    wm = jnp.transpose(w, (2, 3, 1, 0)).reshape(kh * kw, Ci, Co)
    z = jnp.zeros_like(wm)
    rows = []
    for a in range(P):
        rows.append(jnp.concatenate([wm if a == b else z for b in range(P)],
                                    axis=2))
    wb = jnp.concatenate(rows, axis=1)
    return wb.reshape(kh * kw * P * Ci, P * Co).astype(jnp.bfloat16)


@jax.jit
def _forward(x, w1, g1, b1, w2, g2, b2):
    N, C, H, W = x.shape
    HW = H * W
    Co1, Co2 = w1.shape[0], w2.shape[0]
    P = 2 if N % 2 == 0 else 1
    NP = N // P
    grid = (2, NP // 2) if NP % 2 == 0 else (1, NP)
    npc = grid[1]
    params = pltpu.CompilerParams(
        dimension_semantics=("parallel", "arbitrary"),
        vmem_limit_bytes=_VMEM_LIMIT)

    def img(c, i):
        return c * npc + i

    xd = x.reshape(N, C, HW)
    inv_m = 1.0 / (N * HW)
    g1r = g1.astype(jnp.float32).reshape(1, Co1)
    b1r = b1.astype(jnp.float32).reshape(1, Co1)
    g2r = g2.astype(jnp.float32).reshape(1, Co2)
    b2r = b2.astype(jnp.float32).reshape(1, Co2)

    y1, st1 = pl.pallas_call(
        functools.partial(_conv1_kernel, H, W),
        out_shape=(jax.ShapeDtypeStruct((NP, HW, P * Co1), jnp.float32),
                   jax.ShapeDtypeStruct((NP, 8, P * Co1), jnp.float32)),
        grid=grid,
        in_specs=[pl.BlockSpec((P, C, HW), lambda c, i: (img(c, i), 0, 0)),
                  pl.BlockSpec((9 * P * C, P * Co1), lambda c, i: (0, 0))],
        out_specs=(pl.BlockSpec((1, HW, P * Co1),
                                lambda c, i: (img(c, i), 0, 0)),
                   pl.BlockSpec((1, 8, P * Co1),
                                lambda c, i: (img(c, i), 0, 0))),
        compiler_params=params,
    )(xd, _w_block(w1, P))

    y2, st2 = pl.pallas_call(
        functools.partial(_affine_conv2_kernel, P, H, W, inv_m),
        out_shape=(jax.ShapeDtypeStruct((NP, HW, P * Co2), jnp.float32),
                   jax.ShapeDtypeStruct((NP, 8, P * Co2), jnp.float32)),
        grid=grid,
        in_specs=[pl.BlockSpec((1, HW, P * Co1), lambda c, i: (img(c, i), 0, 0)),
                  pl.BlockSpec((NP, 8, P * Co1), lambda c, i: (0, 0, 0)),
                  pl.BlockSpec((1, Co1), lambda c, i: (0, 0)),
                  pl.BlockSpec((1, Co1), lambda c, i: (0, 0)),
                  pl.BlockSpec((9 * P * Co1, P * Co2), lambda c, i: (0, 0))],
        out_specs=(pl.BlockSpec((1, HW, P * Co2),
                                lambda c, i: (img(c, i), 0, 0)),
                   pl.BlockSpec((1, 8, P * Co2),
                                lambda c, i: (img(c, i), 0, 0))),
        compiler_params=params,
    )(y1, st1, g1r, b1r, _w_block(w2, P))

    out = pl.pallas_call(
        functools.partial(_affine_out_kernel, P, H, W, inv_m),
        out_shape=jax.ShapeDtypeStruct((N, Co2, HW), jnp.float32),
        grid=grid,
        in_specs=[pl.BlockSpec((1, HW, P * Co2), lambda c, i: (img(c, i), 0, 0)),
                  pl.BlockSpec((NP, 8, P * Co2), lambda c, i: (0, 0, 0)),
                  pl.BlockSpec((1, Co2), lambda c, i: (0, 0)),
                  pl.BlockSpec((1, Co2), lambda c, i: (0, 0))],
        out_specs=pl.BlockSpec((P, Co2, HW), lambda c, i: (img(c, i), 0, 0)),
        compiler_params=params,
    )(y2, st2, g2r, b2r)

    return out.reshape(N, Co2, H, W)


def kernel(x, w1, g1, b1, w2, g2, b2):
    return _forward(x, w1, g1, b1, w2, g2, b2)
